# baseline (speedup 1.0000x reference)
import functools

import jax
import jax.numpy as jnp
from jax import lax
from jax.experimental import pallas as pl
from jax.experimental.pallas import tpu as pltpu
from jax.experimental.pallas import tpu_sc as plsc

N_CLS = 1000
N_CTX = 16
CTX_DIM = 512
SEQ = 77
SUF = SEQ - 1 - N_CTX
B = 4096

NC = 2
NS = 16
NW = NC * NS
BPW = B // NW

CB = 8


def _table_body(pfx_ref, sfx_ref, ctx_ref, out_ref):
    out_ref[:, 0:1, :] = pfx_ref[...]
    out_ref[:, 1:1 + N_CTX, :] = jnp.broadcast_to(
        ctx_ref[...][None], (CB, N_CTX, CTX_DIM))
    out_ref[:, 1 + N_CTX:SEQ, :] = sfx_ref[...]


_build_table = pl.pallas_call(
    _table_body,
    grid=(N_CLS // CB,),
    in_specs=[
        pl.BlockSpec((CB, 1, CTX_DIM), lambda i: (i, 0, 0)),
        pl.BlockSpec((CB, SUF, CTX_DIM), lambda i: (i, 0, 0)),
        pl.BlockSpec((N_CTX, CTX_DIM), lambda i: (0, 0)),
    ],
    out_specs=pl.BlockSpec((CB, SEQ, CTX_DIM), lambda i: (i, 0, 0)),
    out_shape=jax.ShapeDtypeStruct((N_CLS, SEQ, CTX_DIM), jnp.float32),
)

_mesh = plsc.VectorSubcoreMesh(core_axis_name="c", subcore_axis_name="s")

GRP = 16


@functools.partial(
    pl.kernel,
    out_type=jax.ShapeDtypeStruct((B, SEQ, CTX_DIM), jnp.float32),
    mesh=_mesh,
    scratch_types=[
        pltpu.VMEM((BPW,), jnp.int32),
        pltpu.SemaphoreType.DMA,
    ],
)
def _route_kernel(tbl_hbm, idx_hbm, out_hbm, idx_v, sem):
    wid = lax.axis_index("s") * NC + lax.axis_index("c")
    base = wid * BPW
    pltpu.sync_copy(idx_hbm.at[pl.ds(base, BPW)], idx_v)

    def body(g, carry):
        vec = idx_v[pl.ds(g * GRP, GRP)]
        for lane in range(GRP):
            c = vec[lane]
            pltpu.async_copy(
                tbl_hbm.at[pl.ds(c, 1)],
                out_hbm.at[pl.ds(base + g * GRP + lane, 1)], sem)
        for _ in range(GRP):
            pltpu.make_async_copy(
                tbl_hbm.at[pl.ds(0, 1)],
                out_hbm.at[pl.ds(base, 1)], sem).wait()
        return carry

    lax.fori_loop(0, BPW // GRP, body, 0)


def kernel(ctx, token_prefix, token_suffix, compare_idx):
    tbl = _build_table(token_prefix, token_suffix, ctx)
    idx = compare_idx.astype(jnp.int32)
    return _route_kernel(tbl, idx)


# speedup vs baseline: 18.5103x; 18.5103x over previous
import functools

import jax
import jax.numpy as jnp
from jax import lax
from jax.experimental import pallas as pl
from jax.experimental.pallas import tpu as pltpu
from jax.experimental.pallas import tpu_sc as plsc

N_CLS = 1000
N_CTX = 16
CTX_DIM = 512
SEQ = 77
SUF = SEQ - 1 - N_CTX
B = 4096

NC = 2
NS = 16
NW = NC * NS
BPW = B // NW

CB = 8


def _table_body(pfx_ref, sfx_ref, ctx_ref, out_ref):
    out_ref[:, 0:1, :] = pfx_ref[...]
    out_ref[:, 1:1 + N_CTX, :] = jnp.broadcast_to(
        ctx_ref[...][None], (CB, N_CTX, CTX_DIM))
    out_ref[:, 1 + N_CTX:SEQ, :] = sfx_ref[...]


_build_table = pl.pallas_call(
    _table_body,
    grid=(N_CLS // CB,),
    in_specs=[
        pl.BlockSpec((CB, 1, CTX_DIM), lambda i: (i, 0, 0)),
        pl.BlockSpec((CB, SUF, CTX_DIM), lambda i: (i, 0, 0)),
        pl.BlockSpec((N_CTX, CTX_DIM), lambda i: (0, 0)),
    ],
    out_specs=pl.BlockSpec((CB, SEQ, CTX_DIM), lambda i: (i, 0, 0)),
    out_shape=jax.ShapeDtypeStruct((N_CLS, SEQ, CTX_DIM), jnp.float32),
)

_mesh = plsc.VectorSubcoreMesh(core_axis_name="c", subcore_axis_name="s")

GRP = 16


@functools.partial(
    pl.kernel,
    out_type=jax.ShapeDtypeStruct((B, SEQ, CTX_DIM), jnp.float32),
    mesh=_mesh,
    scratch_types=[
        pltpu.VMEM((BPW,), jnp.int32),
        pltpu.VMEM((1, SEQ, CTX_DIM), jnp.float32),
        pltpu.VMEM((1, SEQ, CTX_DIM), jnp.float32),
        pltpu.SemaphoreType.DMA,
        pltpu.SemaphoreType.DMA,
        pltpu.SemaphoreType.DMA,
        pltpu.SemaphoreType.DMA,
    ],
)
def _route_kernel(tbl_hbm, idx_hbm, out_hbm, idx_v, row0_v, row1_v,
                  semg0, semg1, semw0, semw1):
    wid = lax.axis_index("s") * NC + lax.axis_index("c")
    base = wid * BPW
    pltpu.sync_copy(idx_hbm.at[pl.ds(base, BPW)], idx_v)

    rows = (row0_v, row1_v)
    semgs = (semg0, semg1)
    semws = (semw0, semw1)

    def _fetch(c, s):
        pltpu.async_copy(tbl_hbm.at[pl.ds(c, 1)], rows[s], semgs[s])

    def _fetch_wait(s):
        pltpu.make_async_copy(tbl_hbm.at[pl.ds(0, 1)], rows[s], semgs[s]).wait()

    def _write(i, s):
        pltpu.async_copy(rows[s], out_hbm.at[pl.ds(base + i, 1)], semws[s])

    def _write_wait(s):
        pltpu.make_async_copy(rows[s], out_hbm.at[pl.ds(base, 1)], semws[s]).wait()

    def body(g, carry):
        i0 = g * GRP
        vec = idx_v[pl.ds(i0, GRP)]
        _fetch(vec[0], 0)
        _fetch(vec[1], 1)
        for p in range(GRP // 2):
            _fetch_wait(0)
            _write(i0 + 2 * p, 0)
            _fetch_wait(1)
            _write(i0 + 2 * p + 1, 1)
            _write_wait(0)
            _write_wait(1)
            if p < GRP // 2 - 1:
                _fetch(vec[2 * p + 2], 0)
                _fetch(vec[2 * p + 3], 1)
        return carry

    lax.fori_loop(0, BPW // GRP, body, 0)


def kernel(ctx, token_prefix, token_suffix, compare_idx):
    tbl = _build_table(token_prefix, token_suffix, ctx)
    idx = compare_idx.astype(jnp.int32)
    return _route_kernel(tbl, idx)


# speedup vs baseline: 28.4323x; 1.5360x over previous
import functools

import jax
import jax.numpy as jnp
from jax import lax
from jax.experimental import pallas as pl
from jax.experimental.pallas import tpu as pltpu
from jax.experimental.pallas import tpu_sc as plsc

N_CLS = 1000
N_CTX = 16
CTX_DIM = 512
SEQ = 77
SUF = SEQ - 1 - N_CTX
B = 4096

NC = 2
NS = 16
NW = NC * NS
BPW = B // NW

CB = 8
CHUNK = 64
NT = SEQ * (BPW // CHUNK)
IPW = SEQ * BPW


def _table_body(pfx_ref, sfx_ref, ctx_ref, out_ref):
    for k in range(CB):
        r = k * SEQ
        out_ref[pl.ds(r, 1), :] = pfx_ref[k]
        out_ref[pl.ds(r + 1, N_CTX), :] = ctx_ref[...]
        out_ref[pl.ds(r + 1 + N_CTX, SUF), :] = sfx_ref[k]


_build_table = pl.pallas_call(
    _table_body,
    grid=(N_CLS // CB,),
    in_specs=[
        pl.BlockSpec((CB, 1, CTX_DIM), lambda i: (i, 0, 0)),
        pl.BlockSpec((CB, SUF, CTX_DIM), lambda i: (i, 0, 0)),
        pl.BlockSpec((N_CTX, CTX_DIM), lambda i: (0, 0)),
    ],
    out_specs=pl.BlockSpec((CB * SEQ, CTX_DIM), lambda i: (i, 0)),
    out_shape=jax.ShapeDtypeStruct((N_CLS * SEQ, CTX_DIM), jnp.float32),
)

_mesh = plsc.VectorSubcoreMesh(core_axis_name="c", subcore_axis_name="s")


@functools.partial(
    pl.kernel,
    out_type=jax.ShapeDtypeStruct((SEQ * B, CTX_DIM), jnp.float32),
    mesh=_mesh,
    scratch_types=[
        pltpu.VMEM((IPW,), jnp.int32),
        pltpu.VMEM((CHUNK, CTX_DIM), jnp.float32),
        pltpu.VMEM((CHUNK, CTX_DIM), jnp.float32),
        pltpu.SemaphoreType.DMA,
        pltpu.SemaphoreType.DMA,
        pltpu.SemaphoreType.DMA,
        pltpu.SemaphoreType.DMA,
    ],
)
def _route_kernel(tbl_hbm, eidx_hbm, out_hbm, idx_v, buf0_v, buf1_v,
                  semg0, semg1, semw0, semw1):
    wid = lax.axis_index("s") * NC + lax.axis_index("c")
    base = wid * BPW
    pltpu.sync_copy(eidx_hbm.at[pl.ds(wid * IPW, IPW)], idx_v)

    bufs = (buf0_v, buf1_v)
    semgs = (semg0, semg1)
    semws = (semw0, semw1)

    def _dst(t):
        s = lax.div(t, 2)
        h = t - 2 * s
        return out_hbm.at[pl.ds(s * B + base + h * CHUNK, CHUNK), :]

    def _fetch(t, sl):
        pltpu.async_copy(
            tbl_hbm.at[idx_v.at[pl.ds(t * CHUNK, CHUNK)]], bufs[sl], semgs[sl])

    def _fetch_wait(sl):
        pltpu.make_async_copy(
            tbl_hbm.at[pl.ds(0, CHUNK)], bufs[sl], semgs[sl]).wait()

    def _write(t, sl):
        pltpu.async_copy(bufs[sl], _dst(t), semws[sl])

    def _write_wait(sl):
        pltpu.make_async_copy(bufs[sl], out_hbm.at[pl.ds(0, CHUNK)], semws[sl]).wait()

    _fetch(0, 0)
    _fetch(1, 1)

    def body(m, carry):
        t0 = 2 * m
        t1 = t0 + 1
        _fetch_wait(0)
        _write(t0, 0)
        _fetch_wait(1)
        _write(t1, 1)
        _write_wait(0)
        _fetch(lax.rem(t0 + 2, NT), 0)
        _write_wait(1)
        _fetch(lax.rem(t1 + 2, NT), 1)
        return carry

    lax.fori_loop(0, NT // 2, body, 0)
    _fetch_wait(0)
    _fetch_wait(1)


def kernel(ctx, token_prefix, token_suffix, compare_idx):
    tbl = _build_table(token_prefix, token_suffix, ctx)
    idx = compare_idx.astype(jnp.int32)
    eidx = (idx.reshape(NW, 1, BPW) * SEQ
            + jnp.arange(SEQ, dtype=jnp.int32).reshape(1, SEQ, 1)).reshape(-1)
    out2d = _route_kernel(tbl, eidx)
    return jnp.swapaxes(out2d.reshape(SEQ, B, CTX_DIM), 0, 1)


# speedup vs baseline: 32.9740x; 1.1597x over previous
import functools

import jax
import jax.numpy as jnp
from jax import lax
from jax.experimental import pallas as pl
from jax.experimental.pallas import tpu as pltpu
from jax.experimental.pallas import tpu_sc as plsc

N_CLS = 1000
N_CTX = 16
CTX_DIM = 512
SEQ = 77
SUF = SEQ - 1 - N_CTX
B = 4096

NC = 2
NS = 16
NW = NC * NS
BPW = B // NW

CB = 8
CHUNK = 64
NT = SEQ * (BPW // CHUNK)
IPW = SEQ * BPW


def _table_body(pfx_ref, sfx_ref, ctx_ref, out_ref):
    for k in range(CB):
        r = k * SEQ
        out_ref[pl.ds(r, 1), :] = pfx_ref[k]
        out_ref[pl.ds(r + 1, N_CTX), :] = ctx_ref[...]
        out_ref[pl.ds(r + 1 + N_CTX, SUF), :] = sfx_ref[:, k, :]


_build_table = pl.pallas_call(
    _table_body,
    grid=(N_CLS // CB,),
    in_specs=[
        pl.BlockSpec((CB, 1, CTX_DIM), lambda i: (i, 0, 0)),
        pl.BlockSpec((SUF, CB, CTX_DIM), lambda i: (0, i, 0)),
        pl.BlockSpec((N_CTX, CTX_DIM), lambda i: (0, 0)),
    ],
    out_specs=pl.BlockSpec((CB * SEQ, CTX_DIM), lambda i: (i, 0)),
    out_shape=jax.ShapeDtypeStruct((N_CLS * SEQ, CTX_DIM), jnp.float32),
)

_mesh = plsc.VectorSubcoreMesh(core_axis_name="c", subcore_axis_name="s")


@functools.partial(
    pl.kernel,
    out_type=jax.ShapeDtypeStruct((SEQ * B, CTX_DIM), jnp.float32),
    mesh=_mesh,
    scratch_types=[
        pltpu.VMEM((IPW,), jnp.int32),
        pltpu.VMEM((CHUNK, CTX_DIM), jnp.float32),
        pltpu.VMEM((CHUNK, CTX_DIM), jnp.float32),
        pltpu.SemaphoreType.DMA,
        pltpu.SemaphoreType.DMA,
        pltpu.SemaphoreType.DMA,
        pltpu.SemaphoreType.DMA,
    ],
)
def _route_kernel(tbl_hbm, eidx_hbm, out_hbm, idx_v, buf0_v, buf1_v,
                  semg0, semg1, semw0, semw1):
    wid = lax.axis_index("s") * NC + lax.axis_index("c")
    base = wid * BPW
    pltpu.sync_copy(eidx_hbm.at[pl.ds(wid * IPW, IPW)], idx_v)

    bufs = (buf0_v, buf1_v)
    semgs = (semg0, semg1)
    semws = (semw0, semw1)

    def _dst(t):
        s = lax.div(t, 2)
        h = t - 2 * s
        return out_hbm.at[pl.ds(s * B + base + h * CHUNK, CHUNK), :]

    def _fetch(t, sl):
        pltpu.async_copy(
            tbl_hbm.at[idx_v.at[pl.ds(t * CHUNK, CHUNK)]], bufs[sl], semgs[sl])

    def _fetch_wait(sl):
        pltpu.make_async_copy(
            tbl_hbm.at[pl.ds(0, CHUNK)], bufs[sl], semgs[sl]).wait()

    def _write(t, sl):
        pltpu.async_copy(bufs[sl], _dst(t), semws[sl])

    def _write_wait(sl):
        pltpu.make_async_copy(bufs[sl], out_hbm.at[pl.ds(0, CHUNK)], semws[sl]).wait()

    _fetch(0, 0)
    _fetch(1, 1)

    def body(m, carry):
        t0 = 2 * m
        t1 = t0 + 1
        _fetch_wait(0)
        _write(t0, 0)
        _fetch_wait(1)
        _write(t1, 1)
        _write_wait(0)
        _fetch(lax.rem(t0 + 2, NT), 0)
        _write_wait(1)
        _fetch(lax.rem(t1 + 2, NT), 1)
        return carry

    lax.fori_loop(0, NT // 2, body, 0)
    _fetch_wait(0)
    _fetch_wait(1)


def kernel(ctx, token_prefix, token_suffix, compare_idx):
    tbl = _build_table(token_prefix, jnp.swapaxes(token_suffix, 0, 1), ctx)
    idx = compare_idx.astype(jnp.int32)
    eidx = (idx.reshape(NW, 1, BPW) * SEQ
            + jnp.arange(SEQ, dtype=jnp.int32).reshape(1, SEQ, 1)).reshape(-1)
    out2d = _route_kernel(tbl, eidx)
    return jnp.swapaxes(out2d.reshape(SEQ, B, CTX_DIM), 0, 1)


# speedup vs baseline: 33.8236x; 1.0258x over previous
import functools

import jax
import jax.numpy as jnp
from jax import lax
from jax.experimental import pallas as pl
from jax.experimental.pallas import tpu as pltpu
from jax.experimental.pallas import tpu_sc as plsc

N_CLS = 1000
N_CTX = 16
CTX_DIM = 512
SEQ = 77
SUF = SEQ - 1 - N_CTX
B = 4096

NC = 2
NS = 16
NW = NC * NS
BPW = B // NW

CB = 8
CHUNK = 64
NT = SEQ * (BPW // CHUNK)
IPW = SEQ * BPW


def _table_body(pfx_ref, sfx_ref, ctx_ref, out_ref):
    for k in range(CB):
        r = k * SEQ
        out_ref[pl.ds(r, 1), :] = pfx_ref[k]
        out_ref[pl.ds(r + 1, N_CTX), :] = ctx_ref[...]
        out_ref[pl.ds(r + 1 + N_CTX, SUF), :] = sfx_ref[:, k, :]


_build_table = pl.pallas_call(
    _table_body,
    grid=(N_CLS // CB,),
    in_specs=[
        pl.BlockSpec((CB, 1, CTX_DIM), lambda i: (i, 0, 0)),
        pl.BlockSpec((SUF, CB, CTX_DIM), lambda i: (0, i, 0)),
        pl.BlockSpec((N_CTX, CTX_DIM), lambda i: (0, 0)),
    ],
    out_specs=pl.BlockSpec((CB * SEQ, CTX_DIM), lambda i: (i, 0)),
    out_shape=jax.ShapeDtypeStruct((N_CLS * SEQ, CTX_DIM), jnp.float32),
)

_mesh = plsc.VectorSubcoreMesh(core_axis_name="c", subcore_axis_name="s")


@functools.partial(
    pl.kernel,
    out_type=jax.ShapeDtypeStruct((SEQ * B, CTX_DIM), jnp.float32),
    mesh=_mesh,
    scratch_types=[
        pltpu.VMEM((IPW,), jnp.int32),
        pltpu.VMEM((CHUNK, CTX_DIM), jnp.float32),
        pltpu.VMEM((CHUNK, CTX_DIM), jnp.float32),
        pltpu.VMEM((CHUNK, CTX_DIM), jnp.float32),
        pltpu.SemaphoreType.DMA,
        pltpu.SemaphoreType.DMA,
        pltpu.SemaphoreType.DMA,
        pltpu.SemaphoreType.DMA,
        pltpu.SemaphoreType.DMA,
        pltpu.SemaphoreType.DMA,
    ],
)
def _route_kernel(tbl_hbm, eidx_hbm, out_hbm, idx_v, buf0_v, buf1_v, buf2_v,
                  semg0, semg1, semg2, semw0, semw1, semw2):
    wid = lax.axis_index("s") * NC + lax.axis_index("c")
    base = wid * BPW
    pltpu.sync_copy(eidx_hbm.at[pl.ds(wid * IPW, IPW)], idx_v)

    bufs = (buf0_v, buf1_v, buf2_v)
    semgs = (semg0, semg1, semg2)
    semws = (semw0, semw1, semw2)

    def _dst(t):
        s = lax.div(t, 2)
        h = t - 2 * s
        return out_hbm.at[pl.ds(s * B + base + h * CHUNK, CHUNK), :]

    def _fetch(t, sl):
        pltpu.async_copy(
            tbl_hbm.at[idx_v.at[pl.ds(t * CHUNK, CHUNK)]], bufs[sl], semgs[sl])

    def _fetch_wait(sl):
        pltpu.make_async_copy(
            tbl_hbm.at[pl.ds(0, CHUNK)], bufs[sl], semgs[sl]).wait()

    def _write(t, sl):
        pltpu.async_copy(bufs[sl], _dst(t), semws[sl])

    def _write_wait(sl):
        pltpu.make_async_copy(bufs[sl], out_hbm.at[pl.ds(0, CHUNK)], semws[sl]).wait()

    _fetch(0, 0)
    _fetch(1, 1)
    _fetch(2, 2)

    def body(m, carry):
        t0 = 3 * m
        for j in range(3):
            _fetch_wait(j)
            _write(t0 + j, j)
        for j in range(3):
            _write_wait(j)
            _fetch(lax.rem(t0 + 3 + j, NT), j)
        return carry

    lax.fori_loop(0, NT // 3, body, 0)
    _fetch_wait(0)
    _write(NT - 1, 0)
    _write_wait(0)
    _fetch_wait(1)
    _fetch_wait(2)


def kernel(ctx, token_prefix, token_suffix, compare_idx):
    tbl = _build_table(token_prefix, jnp.swapaxes(token_suffix, 0, 1), ctx)
    idx = compare_idx.astype(jnp.int32)
    eidx = (idx.reshape(NW, 1, BPW) * SEQ
            + jnp.arange(SEQ, dtype=jnp.int32).reshape(1, SEQ, 1)).reshape(-1)
    out2d = _route_kernel(tbl, eidx)
    return jnp.swapaxes(out2d.reshape(SEQ, B, CTX_DIM), 0, 1)


# speedup vs baseline: 36.5926x; 1.0819x over previous
import functools

import jax
import jax.numpy as jnp
from jax import lax
from jax.experimental import pallas as pl
from jax.experimental.pallas import tpu as pltpu
from jax.experimental.pallas import tpu_sc as plsc

N_CLS = 1000
N_CTX = 16
CTX_DIM = 512
SEQ = 77
SUF = SEQ - 1 - N_CTX
B = 4096

NC = 2
NS = 16
NW = NC * NS
BPW = B // NW

CB = 40
CHUNK = 64
NT = SEQ * (BPW // CHUNK)
IPW = SEQ * BPW


def _table_body(pfx_ref, sfx_ref, ctx_ref, out_ref):
    for k in range(CB):
        r = k * SEQ
        out_ref[pl.ds(r, 1), :] = pfx_ref[k]
        out_ref[pl.ds(r + 1, N_CTX), :] = ctx_ref[...]
        out_ref[pl.ds(r + 1 + N_CTX, SUF), :] = sfx_ref[:, k, :]


_build_table = pl.pallas_call(
    _table_body,
    grid=(N_CLS // CB,),
    in_specs=[
        pl.BlockSpec((CB, 1, CTX_DIM), lambda i: (i, 0, 0)),
        pl.BlockSpec((SUF, CB, CTX_DIM), lambda i: (0, i, 0)),
        pl.BlockSpec((N_CTX, CTX_DIM), lambda i: (0, 0)),
    ],
    out_specs=pl.BlockSpec((CB * SEQ, CTX_DIM), lambda i: (i, 0)),
    out_shape=jax.ShapeDtypeStruct((N_CLS * SEQ, CTX_DIM), jnp.float32),
)

_mesh = plsc.VectorSubcoreMesh(core_axis_name="c", subcore_axis_name="s")


@functools.partial(
    pl.kernel,
    out_type=jax.ShapeDtypeStruct((SEQ * B, CTX_DIM), jnp.float32),
    mesh=_mesh,
    scratch_types=[
        pltpu.VMEM((IPW,), jnp.int32),
        pltpu.VMEM((CHUNK, CTX_DIM), jnp.float32),
        pltpu.VMEM((CHUNK, CTX_DIM), jnp.float32),
        pltpu.VMEM((CHUNK, CTX_DIM), jnp.float32),
        pltpu.SemaphoreType.DMA,
        pltpu.SemaphoreType.DMA,
        pltpu.SemaphoreType.DMA,
        pltpu.SemaphoreType.DMA,
        pltpu.SemaphoreType.DMA,
        pltpu.SemaphoreType.DMA,
    ],
)
def _route_kernel(tbl_hbm, eidx_hbm, out_hbm, idx_v, buf0_v, buf1_v, buf2_v,
                  semg0, semg1, semg2, semw0, semw1, semw2):
    wid = lax.axis_index("s") * NC + lax.axis_index("c")
    base = wid * BPW
    pltpu.sync_copy(eidx_hbm.at[pl.ds(wid * IPW, IPW)], idx_v)

    bufs = (buf0_v, buf1_v, buf2_v)
    semgs = (semg0, semg1, semg2)
    semws = (semw0, semw1, semw2)

    def _dst(t):
        s = lax.div(t, 2)
        h = t - 2 * s
        return out_hbm.at[pl.ds(s * B + base + h * CHUNK, CHUNK), :]

    def _fetch(t, sl):
        pltpu.async_copy(
            tbl_hbm.at[idx_v.at[pl.ds(t * CHUNK, CHUNK)]], bufs[sl], semgs[sl])

    def _fetch_wait(sl):
        pltpu.make_async_copy(
            tbl_hbm.at[pl.ds(0, CHUNK)], bufs[sl], semgs[sl]).wait()

    def _write(t, sl):
        pltpu.async_copy(bufs[sl], _dst(t), semws[sl])

    def _write_wait(sl):
        pltpu.make_async_copy(bufs[sl], out_hbm.at[pl.ds(0, CHUNK)], semws[sl]).wait()

    _fetch(0, 0)
    _fetch(1, 1)
    _fetch(2, 2)

    def body(m, carry):
        t0 = 3 * m
        for j in range(3):
            _fetch_wait(j)
            _write(t0 + j, j)
        for j in range(3):
            _write_wait(j)
            _fetch(lax.rem(t0 + 3 + j, NT), j)
        return carry

    lax.fori_loop(0, NT // 3, body, 0)
    _fetch_wait(0)
    _write(NT - 1, 0)
    _write_wait(0)
    _fetch_wait(1)
    _fetch_wait(2)


def kernel(ctx, token_prefix, token_suffix, compare_idx):
    tbl = _build_table(token_prefix, jnp.swapaxes(token_suffix, 0, 1), ctx)
    idx = compare_idx.astype(jnp.int32)
    eidx = (idx.reshape(NW, 1, BPW) * SEQ
            + jnp.arange(SEQ, dtype=jnp.int32).reshape(1, SEQ, 1)).reshape(-1)
    out2d = _route_kernel(tbl, eidx)
    return jnp.swapaxes(out2d.reshape(SEQ, B, CTX_DIM), 0, 1)
